# baseline (speedup 1.0000x reference)
import jax
import jax.numpy as jnp
from jax.experimental import pallas as pl

_T = 512


def _dot(a, b):
    return jax.lax.dot(a.astype(jnp.bfloat16), b.astype(jnp.bfloat16),
                       preferred_element_type=jnp.float32)


def _gnn_body(x1_ref, x2_ref, w0_ref, we0_ref, w1_ref, we1_ref, fp_ref, fc_ref):
    t = pl.program_id(1)
    nt = pl.num_programs(1)
    xa = x1_ref[0]
    xb = x2_ref[0]
    we0 = we0_ref[...]

    pa = _dot(xa, w0_ref[...])
    pb = _dot(xb, w0_ref[...])
    aa, ba, ca, da = pa[:, :128], pa[:, 128:256], pa[:, 256:384], pa[:, 384:]
    ab, bb, cb, db = pb[:, :128], pb[:, 128:256], pb[:, 256:384], pb[:, 384:]
    e_ab = ca + db + we0
    e_ba = cb + da + we0
    ha = jax.nn.relu(aa + jax.nn.sigmoid(e_ba) * bb)
    hb = jax.nn.relu(ab + jax.nn.sigmoid(e_ab) * ba)

    qa = _dot(ha, w1_ref[...])
    qb = _dot(hb, w1_ref[...])
    ea = _dot(e_ab, we1_ref[...])
    eb = _dot(e_ba, we1_ref[...])
    e2_ab = qa[:, 64:96] + qb[:, 96:128] + ea
    e2_ba = qb[:, 64:96] + qa[:, 96:128] + eb
    h2a = jax.nn.relu(qa[:, :32] + jax.nn.sigmoid(e2_ba) * qb[:, 32:64])
    h2b = jax.nn.relu(qb[:, :32] + jax.nn.sigmoid(e2_ab) * qa[:, 32:64])

    inv = jnp.float32(1.0) / jnp.float32(_T * nt)
    fp_part = jnp.concatenate([ha.sum(0), h2a.sum(0)])[None, None, :] * inv
    fc_part = jnp.concatenate([hb.sum(0), h2b.sum(0)])[None, None, :] * inv

    @pl.when(t == 0)
    def _init():
        fp_ref[...] = fp_part
        fc_ref[...] = fc_part

    @pl.when(t != 0)
    def _acc():
        fp_ref[...] += fp_part
        fc_ref[...] += fc_part


def _head_body(fp_ref, fc_ref, ws_ref, bs_ref, wid_ref, bid_ref,
               y_ref, p0_ref, p1_ref, p2_ref, p3_ref, c_ref):
    fp = fp_ref[...]
    fc = fc_ref[...]
    c_ref[...] = 0.5 * (fp + fc)
    y_ref[...] = _dot(jnp.abs(fp - fc), ws_ref[...]) + bs_ref[...]
    f = jnp.concatenate([fp, fc], axis=0)
    for i, p_ref in enumerate((p0_ref, p1_ref, p2_ref, p3_ref)):
        p_ref[...] = _dot(f, wid_ref[i]) + bid_ref[i][None, :]


def kernel(x1_batch, x2_batch, WA0, WB0, WC0, WD0, WE0,
           WA1, WB1, WC1, WD1, WE1, Ws, bs, Wid, bid):
    B, L, d = x1_batch.shape
    d1 = WA1.shape[1]
    feat = d + d1
    out_dim = Ws.shape[1]

    w0 = jnp.concatenate([WA0, WB0, WC0, WD0], axis=1)
    w1 = jnp.concatenate([WA1, WB1, WC1, WD1], axis=1)

    fp, fc = pl.pallas_call(
        _gnn_body,
        grid=(B, L // _T),
        in_specs=[
            pl.BlockSpec((1, _T, d), lambda b, t: (b, t, 0)),
            pl.BlockSpec((1, _T, d), lambda b, t: (b, t, 0)),
            pl.BlockSpec((d, 4 * d), lambda b, t: (0, 0)),
            pl.BlockSpec((1, d), lambda b, t: (0, 0)),
            pl.BlockSpec((d, d), lambda b, t: (0, 0)),
            pl.BlockSpec((d, d1), lambda b, t: (0, 0)),
        ],
        out_specs=[
            pl.BlockSpec((1, 1, feat), lambda b, t: (b, 0, 0)),
            pl.BlockSpec((1, 1, feat), lambda b, t: (b, 0, 0)),
        ],
        out_shape=[
            jax.ShapeDtypeStruct((B, 1, feat), jnp.float32),
            jax.ShapeDtypeStruct((B, 1, feat), jnp.float32),
        ],
    )(x1_batch, x2_batch, w0, WE0, w1, WE1)
    fp = fp.reshape(B, feat)
    fc = fc.reshape(B, feat)

    nj = Wid.shape[1]
    rows = 4 * jnp.arange(nj)[None, :] + jnp.arange(4)[:, None]
    wid_e = jnp.zeros((4, feat, out_dim), jnp.float32).at[
        jnp.arange(4)[:, None], rows].set(Wid)

    y, p0, p1, p2, p3, center = pl.pallas_call(
        _head_body,
        out_shape=[
            jax.ShapeDtypeStruct((B, out_dim), jnp.float32),
            jax.ShapeDtypeStruct((2 * B, out_dim), jnp.float32),
            jax.ShapeDtypeStruct((2 * B, out_dim), jnp.float32),
            jax.ShapeDtypeStruct((2 * B, out_dim), jnp.float32),
            jax.ShapeDtypeStruct((2 * B, out_dim), jnp.float32),
            jax.ShapeDtypeStruct((B, feat), jnp.float32),
        ],
    )(fp, fc, Ws, bs.reshape(1, -1), wid_e, bid)

    return (y, fp, fc, p0, p1, p2, p3, center)


# speedup vs baseline: 1.2175x; 1.2175x over previous
import jax
import jax.numpy as jnp
from jax.experimental import pallas as pl

_T = 2048


def _dot(a, b):
    return jax.lax.dot(a.astype(jnp.bfloat16), b.astype(jnp.bfloat16),
                       preferred_element_type=jnp.float32)


def _gnn_body(x1_ref, x2_ref, w0_ref, we0_ref, w1_ref, we1_ref, fp_ref, fc_ref):
    t = pl.program_id(1)
    nt = pl.num_programs(1)
    xa = x1_ref[0]
    xb = x2_ref[0]
    we0 = we0_ref[...]

    pa = _dot(xa, w0_ref[...])
    pb = _dot(xb, w0_ref[...])
    aa, ba, ca, da = pa[:, :128], pa[:, 128:256], pa[:, 256:384], pa[:, 384:]
    ab, bb, cb, db = pb[:, :128], pb[:, 128:256], pb[:, 256:384], pb[:, 384:]
    e_ab = ca + db + we0
    e_ba = cb + da + we0
    ha = jax.nn.relu(aa + jax.nn.sigmoid(e_ba) * bb)
    hb = jax.nn.relu(ab + jax.nn.sigmoid(e_ab) * ba)

    qa = _dot(ha, w1_ref[...])
    qb = _dot(hb, w1_ref[...])
    ea = _dot(e_ab, we1_ref[...])
    eb = _dot(e_ba, we1_ref[...])
    e2_ab = qa[:, 64:96] + qb[:, 96:128] + ea
    e2_ba = qb[:, 64:96] + qa[:, 96:128] + eb
    h2a = jax.nn.relu(qa[:, :32] + jax.nn.sigmoid(e2_ba) * qb[:, 32:64])
    h2b = jax.nn.relu(qb[:, :32] + jax.nn.sigmoid(e2_ab) * qa[:, 32:64])

    inv = jnp.float32(1.0) / jnp.float32(_T * nt)
    fp_part = jnp.concatenate([ha.sum(0), h2a.sum(0)])[None, None, :] * inv
    fc_part = jnp.concatenate([hb.sum(0), h2b.sum(0)])[None, None, :] * inv

    @pl.when(t == 0)
    def _init():
        fp_ref[...] = fp_part
        fc_ref[...] = fc_part

    @pl.when(t != 0)
    def _acc():
        fp_ref[...] += fp_part
        fc_ref[...] += fc_part


def _head_body(fp_ref, fc_ref, ws_ref, bs_ref, wid_ref, bid_ref,
               y_ref, p0_ref, p1_ref, p2_ref, p3_ref, c_ref):
    fp = fp_ref[...]
    fc = fc_ref[...]
    c_ref[...] = 0.5 * (fp + fc)
    y_ref[...] = _dot(jnp.abs(fp - fc), ws_ref[...]) + bs_ref[...]
    f = jnp.concatenate([fp, fc], axis=0)
    for i, p_ref in enumerate((p0_ref, p1_ref, p2_ref, p3_ref)):
        p_ref[...] = _dot(f, wid_ref[i]) + bid_ref[i][None, :]


def kernel(x1_batch, x2_batch, WA0, WB0, WC0, WD0, WE0,
           WA1, WB1, WC1, WD1, WE1, Ws, bs, Wid, bid):
    B, L, d = x1_batch.shape
    d1 = WA1.shape[1]
    feat = d + d1
    out_dim = Ws.shape[1]

    w0 = jnp.concatenate([WA0, WB0, WC0, WD0], axis=1)
    w1 = jnp.concatenate([WA1, WB1, WC1, WD1], axis=1)

    fp, fc = pl.pallas_call(
        _gnn_body,
        grid=(B, L // _T),
        in_specs=[
            pl.BlockSpec((1, _T, d), lambda b, t: (b, t, 0)),
            pl.BlockSpec((1, _T, d), lambda b, t: (b, t, 0)),
            pl.BlockSpec((d, 4 * d), lambda b, t: (0, 0)),
            pl.BlockSpec((1, d), lambda b, t: (0, 0)),
            pl.BlockSpec((d, d), lambda b, t: (0, 0)),
            pl.BlockSpec((d, d1), lambda b, t: (0, 0)),
        ],
        out_specs=[
            pl.BlockSpec((1, 1, feat), lambda b, t: (b, 0, 0)),
            pl.BlockSpec((1, 1, feat), lambda b, t: (b, 0, 0)),
        ],
        out_shape=[
            jax.ShapeDtypeStruct((B, 1, feat), jnp.float32),
            jax.ShapeDtypeStruct((B, 1, feat), jnp.float32),
        ],
    )(x1_batch, x2_batch, w0, WE0, w1, WE1)
    fp = fp.reshape(B, feat)
    fc = fc.reshape(B, feat)

    nj = Wid.shape[1]
    rows = 4 * jnp.arange(nj)[None, :] + jnp.arange(4)[:, None]
    wid_e = jnp.zeros((4, feat, out_dim), jnp.float32).at[
        jnp.arange(4)[:, None], rows].set(Wid)

    y, p0, p1, p2, p3, center = pl.pallas_call(
        _head_body,
        out_shape=[
            jax.ShapeDtypeStruct((B, out_dim), jnp.float32),
            jax.ShapeDtypeStruct((2 * B, out_dim), jnp.float32),
            jax.ShapeDtypeStruct((2 * B, out_dim), jnp.float32),
            jax.ShapeDtypeStruct((2 * B, out_dim), jnp.float32),
            jax.ShapeDtypeStruct((2 * B, out_dim), jnp.float32),
            jax.ShapeDtypeStruct((B, feat), jnp.float32),
        ],
    )(fp, fc, Ws, bs.reshape(1, -1), wid_e, bid)

    return (y, fp, fc, p0, p1, p2, p3, center)


# speedup vs baseline: 1.3335x; 1.0953x over previous
import jax
import jax.numpy as jnp
from jax.experimental import pallas as pl
from jax.experimental.pallas import tpu as pltpu


def _dot(a, b):
    return jax.lax.dot(a.astype(jnp.bfloat16), b.astype(jnp.bfloat16),
                       preferred_element_type=jnp.float32)


def _body(x1_ref, x2_ref, wa0_ref, wb0_ref, wc0_ref, wd0_ref, we0_ref,
          wa1_ref, wb1_ref, wc1_ref, wd1_ref, we1_ref,
          ws_ref, bs_ref, wid_ref, bid_ref,
          y_ref, fp_ref, fc_ref, p0_ref, p1_ref, p2_ref, p3_ref, c_ref,
          w0s, w1s, we1s, fps, fcs):
    b = pl.program_id(0)
    nb = pl.num_programs(0)

    @pl.when(b == 0)
    def _stage_weights():
        w0s[:, 0:128] = wa0_ref[...].astype(jnp.bfloat16)
        w0s[:, 128:256] = wb0_ref[...].astype(jnp.bfloat16)
        w0s[:, 256:384] = wc0_ref[...].astype(jnp.bfloat16)
        w0s[:, 384:512] = wd0_ref[...].astype(jnp.bfloat16)
        w1s[:, 0:32] = wa1_ref[...].astype(jnp.bfloat16)
        w1s[:, 32:64] = wb1_ref[...].astype(jnp.bfloat16)
        w1s[:, 64:96] = wc1_ref[...].astype(jnp.bfloat16)
        w1s[:, 96:128] = wd1_ref[...].astype(jnp.bfloat16)
        we1s[...] = we1_ref[...].astype(jnp.bfloat16)

    xa = x1_ref[0].astype(jnp.bfloat16)
    xb = x2_ref[0].astype(jnp.bfloat16)
    we0 = we0_ref[...]

    pa = jax.lax.dot(xa, w0s[...], preferred_element_type=jnp.float32)
    pb = jax.lax.dot(xb, w0s[...], preferred_element_type=jnp.float32)
    e_ab = pa[:, 256:384] + pb[:, 384:512] + we0
    e_ba = pb[:, 256:384] + pa[:, 384:512] + we0
    ha = jax.nn.relu(pa[:, 0:128] + jax.nn.sigmoid(e_ba) * pb[:, 128:256])
    hb = jax.nn.relu(pb[:, 0:128] + jax.nn.sigmoid(e_ab) * pa[:, 128:256])
    hab = ha.astype(jnp.bfloat16)
    hbb = hb.astype(jnp.bfloat16)

    qa = jax.lax.dot(hab, w1s[...], preferred_element_type=jnp.float32)
    qb = jax.lax.dot(hbb, w1s[...], preferred_element_type=jnp.float32)
    ea = jax.lax.dot(e_ab.astype(jnp.bfloat16), we1s[...],
                     preferred_element_type=jnp.float32)
    eb = jax.lax.dot(e_ba.astype(jnp.bfloat16), we1s[...],
                     preferred_element_type=jnp.float32)
    e2_ab = qa[:, 64:96] + qb[:, 96:128] + ea
    e2_ba = qb[:, 64:96] + qa[:, 96:128] + eb
    h2a = jax.nn.relu(qa[:, 0:32] + jax.nn.sigmoid(e2_ba) * qb[:, 32:64])
    h2b = jax.nn.relu(qb[:, 0:32] + jax.nn.sigmoid(e2_ab) * qa[:, 32:64])

    t_rows = xa.shape[0]
    inv = jnp.float32(1.0 / t_rows)
    ones8 = jnp.ones((8, t_rows), jnp.bfloat16)
    sa1 = jax.lax.dot(ones8, hab, preferred_element_type=jnp.float32)
    sa2 = jax.lax.dot(ones8, h2a.astype(jnp.bfloat16),
                      preferred_element_type=jnp.float32)
    sb1 = jax.lax.dot(ones8, hbb, preferred_element_type=jnp.float32)
    sb2 = jax.lax.dot(ones8, h2b.astype(jnp.bfloat16),
                      preferred_element_type=jnp.float32)
    nb_rows = fps.shape[0]
    sel1 = jax.lax.broadcasted_iota(jnp.int32, (nb_rows, 128), 0) == b
    sel2 = jax.lax.broadcasted_iota(jnp.int32, (nb_rows, 32), 0) == b
    fps[:, 0:128] = jnp.where(sel1, sa1[0:1] * inv, fps[:, 0:128])
    fps[:, 128:160] = jnp.where(sel2, sa2[0:1] * inv, fps[:, 128:160])
    fcs[:, 0:128] = jnp.where(sel1, sb1[0:1] * inv, fcs[:, 0:128])
    fcs[:, 128:160] = jnp.where(sel2, sb2[0:1] * inv, fcs[:, 128:160])

    @pl.when(b == nb - 1)
    def _head():
        fp = fps[...]
        fc = fcs[...]
        fp_ref[...] = fp
        fc_ref[...] = fc
        c_ref[...] = 0.5 * (fp + fc)
        y_ref[...] = _dot(jnp.abs(fp - fc), ws_ref[...]) + bs_ref[...]
        f = jnp.concatenate([fp, fc], axis=0)
        feat = fp.shape[1]
        nj = feat // 4
        k = jax.lax.broadcasted_iota(jnp.int32, (feat, feat), 0)
        c = jax.lax.broadcasted_iota(jnp.int32, (feat, feat), 1)
        sel = (k == 4 * (c % nj) + c // nj).astype(jnp.bfloat16)
        g = jax.lax.dot(f.astype(jnp.bfloat16), sel,
                        preferred_element_type=jnp.float32)
        for i, p_ref in enumerate((p0_ref, p1_ref, p2_ref, p3_ref)):
            p_ref[...] = _dot(g[:, nj * i:nj * (i + 1)], wid_ref[i]) \
                + bid_ref[i][None, :]


def kernel(x1_batch, x2_batch, WA0, WB0, WC0, WD0, WE0,
           WA1, WB1, WC1, WD1, WE1, Ws, bs, Wid, bid):
    B, L, d = x1_batch.shape
    d1 = WA1.shape[1]
    feat = d + d1
    out_dim = Ws.shape[1]

    y, fp, fc, p0, p1, p2, p3, center = pl.pallas_call(
        _body,
        grid=(B,),
        in_specs=[
            pl.BlockSpec((1, L, d), lambda b: (b, 0, 0)),
            pl.BlockSpec((1, L, d), lambda b: (b, 0, 0)),
            pl.BlockSpec((d, d), lambda b: (0, 0)),
            pl.BlockSpec((d, d), lambda b: (0, 0)),
            pl.BlockSpec((d, d), lambda b: (0, 0)),
            pl.BlockSpec((d, d), lambda b: (0, 0)),
            pl.BlockSpec((1, d), lambda b: (0, 0)),
            pl.BlockSpec((d, d1), lambda b: (0, 0)),
            pl.BlockSpec((d, d1), lambda b: (0, 0)),
            pl.BlockSpec((d, d1), lambda b: (0, 0)),
            pl.BlockSpec((d, d1), lambda b: (0, 0)),
            pl.BlockSpec((d, d1), lambda b: (0, 0)),
            pl.BlockSpec((feat, out_dim), lambda b: (0, 0)),
            pl.BlockSpec((1, out_dim), lambda b: (0, 0)),
            pl.BlockSpec((4, feat // 4, out_dim), lambda b: (0, 0, 0)),
            pl.BlockSpec((4, out_dim), lambda b: (0, 0)),
        ],
        out_specs=[
            pl.BlockSpec((B, out_dim), lambda b: (0, 0)),
            pl.BlockSpec((B, feat), lambda b: (0, 0)),
            pl.BlockSpec((B, feat), lambda b: (0, 0)),
            pl.BlockSpec((2 * B, out_dim), lambda b: (0, 0)),
            pl.BlockSpec((2 * B, out_dim), lambda b: (0, 0)),
            pl.BlockSpec((2 * B, out_dim), lambda b: (0, 0)),
            pl.BlockSpec((2 * B, out_dim), lambda b: (0, 0)),
            pl.BlockSpec((B, feat), lambda b: (0, 0)),
        ],
        out_shape=[
            jax.ShapeDtypeStruct((B, out_dim), jnp.float32),
            jax.ShapeDtypeStruct((B, feat), jnp.float32),
            jax.ShapeDtypeStruct((B, feat), jnp.float32),
            jax.ShapeDtypeStruct((2 * B, out_dim), jnp.float32),
            jax.ShapeDtypeStruct((2 * B, out_dim), jnp.float32),
            jax.ShapeDtypeStruct((2 * B, out_dim), jnp.float32),
            jax.ShapeDtypeStruct((2 * B, out_dim), jnp.float32),
            jax.ShapeDtypeStruct((B, feat), jnp.float32),
        ],
        scratch_shapes=[
            pltpu.VMEM((d, 4 * d), jnp.bfloat16),
            pltpu.VMEM((d, d), jnp.bfloat16),
            pltpu.VMEM((d, d1), jnp.bfloat16),
            pltpu.VMEM((B, feat), jnp.float32),
            pltpu.VMEM((B, feat), jnp.float32),
        ],
    )(x1_batch, x2_batch, WA0, WB0, WC0, WD0, WE0,
      WA1, WB1, WC1, WD1, WE1, Ws, bs.reshape(1, -1), Wid, bid)

    return (y, fp, fc, p0, p1, p2, p3, center)


# speedup vs baseline: 2.0549x; 1.5409x over previous
import jax
import jax.numpy as jnp
from jax.experimental import pallas as pl
from jax.experimental.pallas import tpu as pltpu


def _dot(a, b):
    return jax.lax.dot(a.astype(jnp.bfloat16), b.astype(jnp.bfloat16),
                       preferred_element_type=jnp.float32)


def _body(x1_ref, x2_ref, wa0_ref, wb0_ref, wc0_ref, wd0_ref, we0_ref,
          wa1_ref, wb1_ref, wc1_ref, wd1_ref, we1_ref,
          ws_ref, bs_ref, wid_ref, bid_ref,
          y_ref, fp_ref, fc_ref, p0_ref, p1_ref, p2_ref, p3_ref, c_ref,
          w0s, w1a, w1b, fps, fcs):
    b = pl.program_id(0)
    nb = pl.num_programs(0)

    @pl.when(b == 0)
    def _stage_weights():
        w0s[:, 0:128] = wa0_ref[...].astype(jnp.bfloat16)
        w0s[:, 128:256] = wb0_ref[...].astype(jnp.bfloat16)
        w0s[:, 256:384] = wc0_ref[...].astype(jnp.bfloat16)
        w0s[:, 384:512] = wd0_ref[...].astype(jnp.bfloat16)
        w1a[:, :] = jnp.zeros_like(w1a)
        w1a[0:128, 0:32] = wa1_ref[...].astype(jnp.bfloat16)
        w1a[0:128, 32:64] = wb1_ref[...].astype(jnp.bfloat16)
        w1a[0:128, 64:96] = wd1_ref[...].astype(jnp.bfloat16)
        w1a[0:128, 96:128] = wc1_ref[...].astype(jnp.bfloat16)
        w1a[128:256, 96:128] = we1_ref[...].astype(jnp.bfloat16)
        w1b[:, :] = jnp.zeros_like(w1b)
        w1b[0:128, 0:32] = wb1_ref[...].astype(jnp.bfloat16)
        w1b[0:128, 32:64] = wa1_ref[...].astype(jnp.bfloat16)
        w1b[0:128, 64:96] = wc1_ref[...].astype(jnp.bfloat16)
        w1b[0:128, 96:128] = wd1_ref[...].astype(jnp.bfloat16)
        w1b[128:256, 64:96] = we1_ref[...].astype(jnp.bfloat16)

    xa = x1_ref[0].astype(jnp.bfloat16)
    xb = x2_ref[0].astype(jnp.bfloat16)
    we0 = we0_ref[...]

    pa = jax.lax.dot(xa, w0s[...], preferred_element_type=jnp.float32)
    pb = jax.lax.dot(xb, w0s[...], preferred_element_type=jnp.float32)
    e_ab = pa[:, 256:384] + pb[:, 384:512] + we0
    e_ba = pb[:, 256:384] + pa[:, 384:512] + we0
    ha = jax.nn.relu(pa[:, 0:128] + jax.nn.sigmoid(e_ba) * pb[:, 128:256])
    hb = jax.nn.relu(pb[:, 0:128] + jax.nn.sigmoid(e_ab) * pa[:, 128:256])
    hab = ha.astype(jnp.bfloat16)
    hbb = hb.astype(jnp.bfloat16)

    za = jnp.concatenate([hab, e_ab.astype(jnp.bfloat16)], axis=1)
    zb = jnp.concatenate([hbb, e_ba.astype(jnp.bfloat16)], axis=1)
    qa = jax.lax.dot(za, w1a[...], preferred_element_type=jnp.float32)
    qb = jax.lax.dot(zb, w1b[...], preferred_element_type=jnp.float32)
    sg = jax.nn.sigmoid(qa + qb)
    sgr = jnp.roll(sg, -64, axis=1)
    lane = jax.lax.broadcasted_iota(jnp.int32, qa.shape, 1)
    m0 = lane < 32
    gp = jnp.where(m0, qa, qb)
    gq = jnp.where(m0, qb, qa)
    g2 = jax.nn.relu(gp + sgr * gq)

    t_rows = xa.shape[0]
    inv = jnp.float32(1.0 / t_rows)
    ones8 = jnp.ones((8, t_rows), jnp.bfloat16)
    sa1 = jax.lax.dot(ones8, hab, preferred_element_type=jnp.float32)
    sb1 = jax.lax.dot(ones8, hbb, preferred_element_type=jnp.float32)
    s2 = jax.lax.dot(ones8, g2.astype(jnp.bfloat16),
                     preferred_element_type=jnp.float32)
    nb_rows = fps.shape[0]
    sel1 = jax.lax.broadcasted_iota(jnp.int32, (nb_rows, 128), 0) == b
    sel2 = jax.lax.broadcasted_iota(jnp.int32, (nb_rows, 32), 0) == b
    fps[:, 0:128] = jnp.where(sel1, sa1[0:1] * inv, fps[:, 0:128])
    fps[:, 128:160] = jnp.where(sel2, s2[0:1, 0:32] * inv, fps[:, 128:160])
    fcs[:, 0:128] = jnp.where(sel1, sb1[0:1] * inv, fcs[:, 0:128])
    fcs[:, 128:160] = jnp.where(sel2, s2[0:1, 32:64] * inv, fcs[:, 128:160])

    @pl.when(b == nb - 1)
    def _head():
        fp = fps[...]
        fc = fcs[...]
        fp_ref[...] = fp
        fc_ref[...] = fc
        c_ref[...] = 0.5 * (fp + fc)
        y_ref[...] = _dot(jnp.abs(fp - fc), ws_ref[...]) + bs_ref[...]
        f = jnp.concatenate([fp, fc], axis=0)
        feat = fp.shape[1]
        nj = feat // 4
        k = jax.lax.broadcasted_iota(jnp.int32, (feat, feat), 0)
        c = jax.lax.broadcasted_iota(jnp.int32, (feat, feat), 1)
        sel = (k == 4 * (c % nj) + c // nj).astype(jnp.bfloat16)
        g = jax.lax.dot(f.astype(jnp.bfloat16), sel,
                        preferred_element_type=jnp.float32)
        for i, p_ref in enumerate((p0_ref, p1_ref, p2_ref, p3_ref)):
            p_ref[...] = _dot(g[:, nj * i:nj * (i + 1)], wid_ref[i]) \
                + bid_ref[i][None, :]


def kernel(x1_batch, x2_batch, WA0, WB0, WC0, WD0, WE0,
           WA1, WB1, WC1, WD1, WE1, Ws, bs, Wid, bid):
    B, L, d = x1_batch.shape
    d1 = WA1.shape[1]
    feat = d + d1
    out_dim = Ws.shape[1]

    y, fp, fc, p0, p1, p2, p3, center = pl.pallas_call(
        _body,
        grid=(B,),
        in_specs=[
            pl.BlockSpec((1, L, d), lambda b: (b, 0, 0)),
            pl.BlockSpec((1, L, d), lambda b: (b, 0, 0)),
            pl.BlockSpec((d, d), lambda b: (0, 0)),
            pl.BlockSpec((d, d), lambda b: (0, 0)),
            pl.BlockSpec((d, d), lambda b: (0, 0)),
            pl.BlockSpec((d, d), lambda b: (0, 0)),
            pl.BlockSpec((1, d), lambda b: (0, 0)),
            pl.BlockSpec((d, d1), lambda b: (0, 0)),
            pl.BlockSpec((d, d1), lambda b: (0, 0)),
            pl.BlockSpec((d, d1), lambda b: (0, 0)),
            pl.BlockSpec((d, d1), lambda b: (0, 0)),
            pl.BlockSpec((d, d1), lambda b: (0, 0)),
            pl.BlockSpec((feat, out_dim), lambda b: (0, 0)),
            pl.BlockSpec((1, out_dim), lambda b: (0, 0)),
            pl.BlockSpec((4, feat // 4, out_dim), lambda b: (0, 0, 0)),
            pl.BlockSpec((4, out_dim), lambda b: (0, 0)),
        ],
        out_specs=[
            pl.BlockSpec((B, out_dim), lambda b: (0, 0)),
            pl.BlockSpec((B, feat), lambda b: (0, 0)),
            pl.BlockSpec((B, feat), lambda b: (0, 0)),
            pl.BlockSpec((2 * B, out_dim), lambda b: (0, 0)),
            pl.BlockSpec((2 * B, out_dim), lambda b: (0, 0)),
            pl.BlockSpec((2 * B, out_dim), lambda b: (0, 0)),
            pl.BlockSpec((2 * B, out_dim), lambda b: (0, 0)),
            pl.BlockSpec((B, feat), lambda b: (0, 0)),
        ],
        out_shape=[
            jax.ShapeDtypeStruct((B, out_dim), jnp.float32),
            jax.ShapeDtypeStruct((B, feat), jnp.float32),
            jax.ShapeDtypeStruct((B, feat), jnp.float32),
            jax.ShapeDtypeStruct((2 * B, out_dim), jnp.float32),
            jax.ShapeDtypeStruct((2 * B, out_dim), jnp.float32),
            jax.ShapeDtypeStruct((2 * B, out_dim), jnp.float32),
            jax.ShapeDtypeStruct((2 * B, out_dim), jnp.float32),
            jax.ShapeDtypeStruct((B, feat), jnp.float32),
        ],
        scratch_shapes=[
            pltpu.VMEM((d, 4 * d), jnp.bfloat16),
            pltpu.VMEM((2 * d, d), jnp.bfloat16),
            pltpu.VMEM((2 * d, d), jnp.bfloat16),
            pltpu.VMEM((B, feat), jnp.float32),
            pltpu.VMEM((B, feat), jnp.float32),
        ],
    )(x1_batch, x2_batch, WA0, WB0, WC0, WD0, WE0,
      WA1, WB1, WC1, WD1, WE1, Ws, bs.reshape(1, -1), Wid, bid)

    return (y, fp, fc, p0, p1, p2, p3, center)
